# baseline (speedup 1.0000x reference)
import jax
import jax.numpy as jnp
from jax.experimental import pallas as pl
from jax.experimental.pallas import tpu as pltpu

N = 4096
D_IN = 512
H1 = 256
H2 = 128
NZ = 16
NC = 10
ALPHA = 0.2
V = 1.0
A = 0.5

BI = 256
GRID = N // BI
NEG = -9e15

_f32 = jnp.float32
_bf16 = jnp.bfloat16


def _relu(v):
    return jnp.maximum(v, 0.0)


def _leaky(v):
    return jnp.maximum(v, ALPHA * v)


def _elu(v):
    return jnp.where(v > 0, v, jnp.exp(jnp.minimum(v, 0.0)) - 1.0)


def _dot(a, b):
    return jnp.dot(a, b, preferred_element_type=jnp.float32)


def _dot_t(a, b):
    return jax.lax.dot_general(a, b, (((1,), (1,)), ((), ())),
                               preferred_element_type=jnp.float32)


def _attention(adj_blk, h_full, h_rows, a_s, a_n):
    s_row = _dot(h_rows, a_s)
    t_all = _dot(h_full, a_n)
    m = _leaky(s_row + jnp.max(t_all))
    e = _leaky(s_row + t_all.T)
    att = jnp.where(adj_blk > 0, e, NEG)
    p = jnp.exp(att - m)
    l = jnp.sum(p, axis=1, keepdims=True)
    l = jnp.maximum(l, 1e-30)
    y = _dot(p, h_full) / l
    return _elu(y)


def _struct_loss_partial(neg_g_row, g_full, adj_blk):
    u = jnp.exp(_dot_t(neg_g_row, g_full))
    d = 1.0 / (1.0 + u) - adj_blk
    return jnp.sum(d * d)


def _accum_loss(loss_ref, partial, first):
    p11 = jnp.reshape(partial, (1, 1))

    @pl.when(first)
    def _():
        loss_ref[...] = p11

    @pl.when(jnp.logical_not(first))
    def _():
        loss_ref[...] += p11


def _k0(x_ref, we1, be1, we2, be2, wz, bz, wd1, bd1, wd2, bd2, wxb, bxb, wg1,
        xbar_ref, zae_ref, eh1_ref, eh2_ref, h1_ref):
    x = x_ref[...]
    eh1 = _relu(_dot(x, we1[...]) + be1[...])
    eh2 = _relu(_dot(eh1, we2[...]) + be2[...])
    zae = _dot(eh2, wz[...]) + bz[...]
    dh1 = _relu(_dot(zae, wd1[...]) + bd1[...])
    dh2 = _relu(_dot(dh1, wd2[...]) + bd2[...])
    xbar_ref[...] = _dot(dh2, wxb[...]) + bxb[...]
    zae_ref[...] = zae
    eh1_ref[...] = eh1
    eh2_ref[...] = eh2
    h1_ref[...] = _dot(x, wg1[...])


def _m(adj_ref, h1_ref, eh1_ref, eh2_ref, zae_ref,
       as1, an1, as2, an2, as3, an3, wg2, wg3,
       ahat_ref, zl_ref, l0_ref, l1_ref, l2_ref,
       g1b_s, h2_s, g2b_s, sm_s):
    p = pl.program_id(0)
    i = pl.program_id(1)
    adj_blk = adj_ref[...]
    rows = pl.ds(i * BI, BI)

    @pl.when(p == 0)
    def _():
        g1 = _attention(adj_blk, h1_ref[...], h1_ref[rows, :],
                        as1[...], an1[...])
        g1b_s[rows, :] = g1.astype(_bf16)
        xin2 = (1.0 - A) * g1 + A * eh1_ref[...]
        h2_s[rows, :] = _dot(xin2, wg2[...])

    @pl.when(p == 1)
    def _():
        g2 = _attention(adj_blk, h2_s[...], h2_s[rows, :],
                        as2[...], an2[...])
        g2b_s[rows, :] = g2.astype(_bf16)
        xin3 = (1.0 - A) * g2 + A * eh2_ref[...]
        sm_s[rows, 0:NZ] = _dot(xin3, wg3[...])
        partial = _struct_loss_partial(-g1b_s[rows, :], g1b_s[...], adj_blk)
        _accum_loss(l0_ref, partial, i == 0)

    @pl.when(p == 2)
    def _():
        zg = _attention(adj_blk, sm_s[:, 0:NZ], sm_s[rows, 0:NZ],
                        as3[...], an3[...])
        sm_s[rows, NZ:2 * NZ] = zg
        sm_s[rows, 2 * NZ:3 * NZ] = (1.0 - A) * zg + A * zae_ref[...]
        partial = _struct_loss_partial(-g2b_s[rows, :], g2b_s[...], adj_blk)
        _accum_loss(l1_ref, partial, i == 0)

    @pl.when(p == 3)
    def _():
        u = jnp.exp(_dot_t(-sm_s[rows, NZ:2 * NZ], sm_s[:, NZ:2 * NZ]))
        ah = 1.0 / (1.0 + u)
        ahat_ref[...] = ah
        d = ah - adj_blk
        _accum_loss(l2_ref, jnp.sum(d * d), i == 0)
        zl_ref[...] = _dot(adj_blk, sm_s[:, 2 * NZ:3 * NZ])


def _soft_assign(z, cluster):
    zn = jnp.sum(z * z, axis=1, keepdims=True)
    cn = jnp.sum(cluster * cluster, axis=1, keepdims=True)
    d2 = zn - 2.0 * _dot_t(z, cluster) + cn.T
    q = 1.0 / (1.0 + d2 / V)
    return q / jnp.sum(q, axis=1, keepdims=True)


def _kq(zl_ref, zae_ref, cl_ref, l0_ref, l1_ref, l2_ref,
        q_ref, q1_ref, tot_ref):
    cl = cl_ref[...]
    q_ref[...] = _soft_assign(zl_ref[...], cl)
    q1_ref[...] = _soft_assign(zae_ref[...], cl)
    scale = 1.0 / (N * N)
    tot_ref[...] = (l0_ref[...] + l1_ref[...] + l2_ref[...]) * scale


_SEQ2 = pltpu.CompilerParams(dimension_semantics=("arbitrary", "arbitrary"))


def kernel(x, adj, W_e1, b_e1, W_e2, b_e2, W_z, b_z, W_d1, b_d1, W_d2, b_d2,
           W_xb, b_xb, Wg1, as1, an1, Wg2, as2, an2, Wg3, as3, an3,
           cluster_layer):
    f = _f32
    b2 = lambda b: b.reshape(1, -1)

    xbar, zae, eh1, eh2, h1 = pl.pallas_call(
        _k0,
        out_shape=[
            jax.ShapeDtypeStruct((N, D_IN), f),
            jax.ShapeDtypeStruct((N, NZ), f),
            jax.ShapeDtypeStruct((N, H1), f),
            jax.ShapeDtypeStruct((N, H2), f),
            jax.ShapeDtypeStruct((N, H1), f),
        ],
    )(x, W_e1, b2(b_e1), W_e2, b2(b_e2), W_z, b2(b_z), W_d1, b2(b_d1),
      W_d2, b2(b_d2), W_xb, b2(b_xb), Wg1)

    full = lambda *shape: pl.BlockSpec(shape, lambda p, i: (0,) * len(shape))
    rows_at = lambda width, phase: pl.BlockSpec(
        (BI, width), lambda p, i, ph=phase: (jnp.where(p == ph, i, 0), 0))
    adj_rows = pl.BlockSpec((BI, N), lambda p, i: (i, 0))
    scal = pl.BlockSpec((1, 1), lambda p, i: (0, 0))

    ahat, zl, l0, l1, l2 = pl.pallas_call(
        _m,
        grid=(4, GRID),
        in_specs=[adj_rows,
                  full(N, H1),
                  rows_at(H1, 0),
                  rows_at(H2, 1),
                  rows_at(NZ, 2),
                  full(H1, 1), full(H1, 1),
                  full(H2, 1), full(H2, 1),
                  full(NZ, 1), full(NZ, 1),
                  full(H1, H2), full(H2, NZ)],
        out_specs=[rows_at(N, 3), rows_at(NZ, 3), scal, scal, scal],
        out_shape=[jax.ShapeDtypeStruct((N, N), f),
                   jax.ShapeDtypeStruct((N, NZ), f),
                   jax.ShapeDtypeStruct((1, 1), f),
                   jax.ShapeDtypeStruct((1, 1), f),
                   jax.ShapeDtypeStruct((1, 1), f)],
        scratch_shapes=[pltpu.VMEM((N, H1), _bf16),
                        pltpu.VMEM((N, H2), _f32),
                        pltpu.VMEM((N, H2), _bf16),
                        pltpu.VMEM((N, 8 * NZ), _f32)],
        compiler_params=_SEQ2,
    )(adj, h1, eh1, eh2, zae, as1, an1, as2, an2, as3, an3, Wg2, Wg3)

    q, q1, tot = pl.pallas_call(
        _kq,
        out_shape=[jax.ShapeDtypeStruct((N, NC), f),
                   jax.ShapeDtypeStruct((N, NC), f),
                   jax.ShapeDtypeStruct((1, 1), f)],
    )(zl, zae, cluster_layer, l0, l1, l2)

    return (xbar, ahat, zae, q, q1, zl, tot.reshape(()))
